# Initial kernel scaffold; baseline (speedup 1.0000x reference)
import jax
import jax.numpy as jnp
from jax.experimental import pallas as pl


def kernel(x, edge_index, W_neigh1, W_self1, b_self1, W_neigh2, W_self2, b_self2):
    raise NotImplementedError("write your pallas kernel here")


# speedup vs baseline: 4.7807x; 4.7807x over previous
import jax
import jax.numpy as jnp
from jax import lax
from jax.experimental import pallas as pl
from jax.experimental.pallas import tpu as pltpu
from jax.experimental.pallas import tpu_sc as plsc
import functools

N = 10000
E = 320000
D = 128

NC = 2
NS = 16
NW = NC * NS
E_PER_TILE = E // NW
CHUNK = 80
N_CHUNKS = E_PER_TILE // CHUNK
NP = 10240
ROWS_PER_TILE = NP // NS


def _sc_agg_body(x_hbm, src_hbm, dst_hbm, acc_hbm, sh_acc, src_v, dst_v,
                 rows_v, sem):
    c = lax.axis_index("c")
    s = lax.axis_index("s")
    wid = c * NS + s
    row0 = s * ROWS_PER_TILE

    def zero_rows(t, carry):
        rows_v[t // 8, pl.ds((t % 8) * 16, 16)] = jnp.zeros((16,), jnp.float32)
        return carry
    lax.fori_loop(0, CHUNK * (D // 16), zero_rows, None)
    for r in range(ROWS_PER_TILE // CHUNK):
        pltpu.sync_copy(rows_v, sh_acc.at[pl.ds(row0 + r * CHUNK, CHUNK)])
    plsc.subcore_barrier()

    e0 = wid * E_PER_TILE

    def edge_step(i, carry):
        base = e0 + i * CHUNK
        pltpu.sync_copy(src_hbm.at[pl.ds(base, CHUNK)], src_v)
        pltpu.sync_copy(dst_hbm.at[pl.ds(base, CHUNK)], dst_v)
        pltpu.async_copy(x_hbm.at[src_v], rows_v, sem).wait()
        pltpu.sync_copy(rows_v, sh_acc.at[dst_v], add=True)
        return carry

    lax.fori_loop(0, N_CHUNKS, edge_step, None)
    plsc.subcore_barrier()

    pltpu.sync_copy(sh_acc.at[pl.ds(row0, ROWS_PER_TILE)],
                    acc_hbm.at[c, pl.ds(row0, ROWS_PER_TILE)])


def _sc_cnt_body(dst_hbm, cnt_hbm, sh_cnt, dst_v, ones_v, sem):
    c = lax.axis_index("c")
    s = lax.axis_index("s")
    wid = c * NS + s
    row0 = s * ROWS_PER_TILE

    def zero_rows(t, carry):
        ones_v[t // 8, pl.ds((t % 8) * 16, 16)] = jnp.zeros((16,), jnp.float32)
        return carry
    lax.fori_loop(0, CHUNK * (D // 16), zero_rows, None)
    for r in range(ROWS_PER_TILE // CHUNK):
        pltpu.sync_copy(ones_v, sh_cnt.at[pl.ds(row0 + r * CHUNK, CHUNK)])

    def fill_rows(t, carry):
        ones_v[t // 8, pl.ds((t % 8) * 16, 16)] = jnp.ones((16,), jnp.float32)
        return carry
    lax.fori_loop(0, CHUNK * (D // 16), fill_rows, None)
    plsc.subcore_barrier()

    e0 = wid * E_PER_TILE

    def edge_step(i, carry):
        base = e0 + i * CHUNK
        pltpu.sync_copy(dst_hbm.at[pl.ds(base, CHUNK)], dst_v)
        pltpu.sync_copy(ones_v, sh_cnt.at[dst_v], add=True)
        return carry

    lax.fori_loop(0, N_CHUNKS, edge_step, None)
    plsc.subcore_barrier()

    pltpu.sync_copy(sh_cnt.at[pl.ds(row0, ROWS_PER_TILE)],
                    cnt_hbm.at[c, pl.ds(row0, ROWS_PER_TILE)])


_sc_cache = {}


def _get_sc_agg():
    if "agg" not in _sc_cache:
        mesh = plsc.VectorSubcoreMesh(core_axis_name="c", subcore_axis_name="s")
        _sc_cache["agg"] = pl.kernel(
            _sc_agg_body,
            out_type=jax.ShapeDtypeStruct((NC, NP, D), jnp.float32),
            mesh=mesh,
            scratch_types=[
                pltpu.VMEM_SHARED((NP, D), jnp.float32),
                pltpu.VMEM((CHUNK,), jnp.int32),
                pltpu.VMEM((CHUNK,), jnp.int32),
                pltpu.VMEM((CHUNK, D), jnp.float32),
                pltpu.SemaphoreType.DMA,
            ],
        )
    return _sc_cache["agg"]


def _get_sc_cnt():
    if "cnt" not in _sc_cache:
        mesh = plsc.VectorSubcoreMesh(core_axis_name="c", subcore_axis_name="s")
        _sc_cache["cnt"] = pl.kernel(
            _sc_cnt_body,
            out_type=jax.ShapeDtypeStruct((NC, NP, D), jnp.float32),
            mesh=mesh,
            scratch_types=[
                pltpu.VMEM_SHARED((NP, D), jnp.float32),
                pltpu.VMEM((CHUNK,), jnp.int32),
                pltpu.VMEM((CHUNK, D), jnp.float32),
                pltpu.SemaphoreType.DMA,
            ],
        )
    return _sc_cache["cnt"]


def _tc_layer_body(relu, x_ref, a_ref, c_ref, wn_ref, ws_ref, b_ref, o_ref):
    cnt = c_ref[0, :, 0:1] + c_ref[1, :, 0:1]
    scale = 1.0 / jnp.maximum(cnt, 1.0)
    neigh = (a_ref[0] + a_ref[1]) * scale
    dn = (((1,), (1,)), ((), ()))
    out = (lax.dot_general(x_ref[...], ws_ref[...], dn,
                           preferred_element_type=jnp.float32)
           + b_ref[...]
           + lax.dot_general(neigh, wn_ref[...], dn,
                             preferred_element_type=jnp.float32))
    if relu:
        out = jnp.maximum(out, 0.0)
    o_ref[...] = out


def _tc_layer(x, acc, cnt, Wn, Ws, b, relu):
    BN = 1000
    grid = (N // BN,)
    return pl.pallas_call(
        functools.partial(_tc_layer_body, relu),
        grid=grid,
        in_specs=[
            pl.BlockSpec((BN, D), lambda i: (i, 0)),
            pl.BlockSpec((NC, BN, D), lambda i: (0, i, 0)),
            pl.BlockSpec((NC, BN, D), lambda i: (0, i, 0)),
            pl.BlockSpec((D, D), lambda i: (0, 0)),
            pl.BlockSpec((D, D), lambda i: (0, 0)),
            pl.BlockSpec((1, D), lambda i: (0, 0)),
        ],
        out_specs=pl.BlockSpec((BN, D), lambda i: (i, 0)),
        out_shape=jax.ShapeDtypeStruct((N, D), jnp.float32),
    )(x, acc, cnt, Wn, Ws, b)


def kernel(x, edge_index, W_neigh1, W_self1, b_self1, W_neigh2, W_self2, b_self2):
    src = edge_index[0]
    dst = edge_index[1]
    cnt = _get_sc_cnt()(dst)
    acc1 = _get_sc_agg()(x, src, dst)
    h = _tc_layer(x, acc1, cnt, W_neigh1, W_self1, b_self1.reshape(1, D), True)
    acc2 = _get_sc_agg()(h, src, dst)
    out = _tc_layer(h, acc2, cnt, W_neigh2, W_self2, b_self2.reshape(1, D), False)
    return out


# speedup vs baseline: 8.6801x; 1.8157x over previous
import jax
import jax.numpy as jnp
from jax import lax
from jax.experimental import pallas as pl
from jax.experimental.pallas import tpu as pltpu
from jax.experimental.pallas import tpu_sc as plsc
import functools

N = 10000
E = 320000
D = 128

NC = 2
NS = 16
NW = NC * NS
E_PER_TILE = E // NW
CHUNK = 80
N_CHUNKS = E_PER_TILE // CHUNK
NP = 10240
ROWS_PER_TILE = NP // NS


SUP = 25
NSUP = N_CHUNKS // SUP


def _sc_agg_body(x_hbm, src_hbm, dst2_hbm, acc_hbm, sh_acc, sidx, didx,
                 rows_a, rows_b, sem_a, sem_b):
    c = lax.axis_index("c")
    s = lax.axis_index("s")
    wid = c * NS + s
    row0 = s * ROWS_PER_TILE

    def zero_rows(t, carry):
        rows_a[t // 8, pl.ds((t % 8) * 16, 16)] = jnp.zeros((16,), jnp.float32)
        return carry
    lax.fori_loop(0, CHUNK * (D // 16), zero_rows, None)
    for r in range(ROWS_PER_TILE // CHUNK):
        pltpu.sync_copy(rows_a, sh_acc.at[pl.ds(row0 + r * CHUNK, CHUNK)])
    plsc.subcore_barrier()

    e0 = wid * E_PER_TILE

    def gather(j, buf, sem):
        pltpu.async_copy(x_hbm.at[sidx.at[pl.ds(j * CHUNK, CHUNK)]], buf, sem)

    def gwait(buf, sem):
        pltpu.make_async_copy(x_hbm.at[pl.ds(0, CHUNK)], buf, sem).wait()

    def scat(j, buf):
        pltpu.sync_copy(buf, sh_acc.at[didx.at[j]], add=True)

    for sp in range(NSUP):
        s0 = e0 + sp * SUP * CHUNK
        pltpu.sync_copy(src_hbm.at[pl.ds(s0, SUP * CHUNK)], sidx)
        pltpu.sync_copy(dst2_hbm.at[wid * NSUP + sp], didx)
        gather(0, rows_a, sem_a)
        gather(1, rows_b, sem_b)

        def pair(t, carry):
            j0 = 2 * t
            gwait(rows_a, sem_a)
            scat(j0, rows_a)
            gather(j0 + 2, rows_a, sem_a)
            gwait(rows_b, sem_b)
            scat(j0 + 1, rows_b)
            gather(j0 + 3, rows_b, sem_b)
            return carry

        lax.fori_loop(0, (SUP - 3) // 2, pair, None)
        gwait(rows_a, sem_a)
        scat(SUP - 3, rows_a)
        gather(SUP - 1, rows_a, sem_a)
        gwait(rows_b, sem_b)
        scat(SUP - 2, rows_b)
        gwait(rows_a, sem_a)
        scat(SUP - 1, rows_a)

    plsc.subcore_barrier()

    pltpu.sync_copy(sh_acc.at[pl.ds(row0, ROWS_PER_TILE)],
                    acc_hbm.at[c, pl.ds(row0, ROWS_PER_TILE)])


def _sc_cnt_body(dst_hbm, cnt_hbm, sh_cnt, dst_v, ones_v, sem):
    c = lax.axis_index("c")
    s = lax.axis_index("s")
    wid = c * NS + s
    row0 = s * ROWS_PER_TILE

    def zero_rows(t, carry):
        ones_v[t // 8, pl.ds((t % 8) * 16, 16)] = jnp.zeros((16,), jnp.float32)
        return carry
    lax.fori_loop(0, CHUNK * (D // 16), zero_rows, None)
    for r in range(ROWS_PER_TILE // CHUNK):
        pltpu.sync_copy(ones_v, sh_cnt.at[pl.ds(row0 + r * CHUNK, CHUNK)])

    def fill_rows(t, carry):
        ones_v[t // 8, pl.ds((t % 8) * 16, 16)] = jnp.ones((16,), jnp.float32)
        return carry
    lax.fori_loop(0, CHUNK * (D // 16), fill_rows, None)
    plsc.subcore_barrier()

    e0 = wid * E_PER_TILE

    def edge_step(i, carry):
        base = e0 + i * CHUNK
        pltpu.sync_copy(dst_hbm.at[pl.ds(base, CHUNK)], dst_v)
        pltpu.sync_copy(ones_v, sh_cnt.at[dst_v], add=True)
        return carry

    lax.fori_loop(0, N_CHUNKS, edge_step, None)
    plsc.subcore_barrier()

    pltpu.sync_copy(sh_cnt.at[pl.ds(row0, ROWS_PER_TILE)],
                    cnt_hbm.at[c, pl.ds(row0, ROWS_PER_TILE)])


_sc_cache = {}


def _get_sc_agg():
    if "agg" not in _sc_cache:
        mesh = plsc.VectorSubcoreMesh(core_axis_name="c", subcore_axis_name="s")
        _sc_cache["agg"] = pl.kernel(
            _sc_agg_body,
            out_type=jax.ShapeDtypeStruct((NC, NP, D), jnp.float32),
            mesh=mesh,
            scratch_types=[
                pltpu.VMEM_SHARED((NP, D), jnp.float32),
                pltpu.VMEM((SUP * CHUNK,), jnp.int32),
                pltpu.VMEM((SUP, CHUNK), jnp.int32),
                pltpu.VMEM((CHUNK, D), jnp.float32),
                pltpu.VMEM((CHUNK, D), jnp.float32),
                pltpu.SemaphoreType.DMA,
                pltpu.SemaphoreType.DMA,
            ],
        )
    return _sc_cache["agg"]


def _get_sc_cnt():
    if "cnt" not in _sc_cache:
        mesh = plsc.VectorSubcoreMesh(core_axis_name="c", subcore_axis_name="s")
        _sc_cache["cnt"] = pl.kernel(
            _sc_cnt_body,
            out_type=jax.ShapeDtypeStruct((NC, NP, D), jnp.float32),
            mesh=mesh,
            scratch_types=[
                pltpu.VMEM_SHARED((NP, D), jnp.float32),
                pltpu.VMEM((CHUNK,), jnp.int32),
                pltpu.VMEM((CHUNK, D), jnp.float32),
                pltpu.SemaphoreType.DMA,
            ],
        )
    return _sc_cache["cnt"]


def _tc_layer_body(relu, x_ref, a_ref, c_ref, wn_ref, ws_ref, b_ref, o_ref):
    cnt = c_ref[0, :, 0:1] + c_ref[1, :, 0:1]
    scale = 1.0 / jnp.maximum(cnt, 1.0)
    neigh = (a_ref[0] + a_ref[1]) * scale
    dn = (((1,), (1,)), ((), ()))
    out = (lax.dot_general(x_ref[...], ws_ref[...], dn,
                           preferred_element_type=jnp.float32)
           + b_ref[...]
           + lax.dot_general(neigh, wn_ref[...], dn,
                             preferred_element_type=jnp.float32))
    if relu:
        out = jnp.maximum(out, 0.0)
    o_ref[...] = out


def _tc_layer(x, acc, cnt, Wn, Ws, b, relu):
    BN = 1000
    grid = (N // BN,)
    return pl.pallas_call(
        functools.partial(_tc_layer_body, relu),
        grid=grid,
        in_specs=[
            pl.BlockSpec((BN, D), lambda i: (i, 0)),
            pl.BlockSpec((NC, BN, D), lambda i: (0, i, 0)),
            pl.BlockSpec((NC, BN, D), lambda i: (0, i, 0)),
            pl.BlockSpec((D, D), lambda i: (0, 0)),
            pl.BlockSpec((D, D), lambda i: (0, 0)),
            pl.BlockSpec((1, D), lambda i: (0, 0)),
        ],
        out_specs=pl.BlockSpec((BN, D), lambda i: (i, 0)),
        out_shape=jax.ShapeDtypeStruct((N, D), jnp.float32),
    )(x, acc, cnt, Wn, Ws, b)


def kernel(x, edge_index, W_neigh1, W_self1, b_self1, W_neigh2, W_self2, b_self2):
    src = edge_index[0]
    dst = edge_index[1]
    dst2 = dst.reshape(NW * NSUP, SUP, CHUNK)
    cnt = _get_sc_cnt()(dst)
    acc1 = _get_sc_agg()(x, src, dst2)
    h = _tc_layer(x, acc1, cnt, W_neigh1, W_self1, b_self1.reshape(1, D), True)
    acc2 = _get_sc_agg()(h, src, dst2)
    out = _tc_layer(h, acc2, cnt, W_neigh2, W_self2, b_self2.reshape(1, D), False)
    return out
